# Initial kernel scaffold; baseline (speedup 1.0000x reference)
import jax
import jax.numpy as jnp
from jax.experimental import pallas as pl


def kernel(schema_params, y_indices, action_to_schema):
    raise NotImplementedError("write your pallas kernel here")


# speedup vs baseline: 3.1493x; 3.1493x over previous
import functools

import jax
import jax.numpy as jnp
from jax import lax
from jax.experimental import pallas as pl
from jax.experimental.pallas import tpu as pltpu
from jax.experimental.pallas import tpu_sc as plsc

B, P, S, L = 1024, 16384, 8, 32
NC, NS = 2, 16
NW = NC * NS
ROWS = B // NW
LANES = 16


def _sc_body(params_hbm, y_hbm, a2s_hbm, pre_hbm, add_hbm, del_hbm,
             y_v, a2s_v, idx_v, prow_v, bpre_v, badd_v, bdel_v, gsem, osem):
    wid = lax.axis_index("s") * NC + lax.axis_index("c")
    base = wid * ROWS

    pltpu.sync_copy(y_hbm.at[pl.ds(base * L, ROWS * L)], y_v)
    pltpu.sync_copy(a2s_hbm.at[pl.ds(base, ROWS)], a2s_v)

    zf = jnp.zeros((LANES,), jnp.float32)
    iota = lax.iota(jnp.int32, LANES)

    def _zero(i, _):
        bpre_v[pl.ds(i * LANES, LANES)] = zf
        badd_v[pl.ds(i * LANES, LANES)] = zf
        bdel_v[pl.ds(i * LANES, LANES)] = zf
        return 0

    lax.fori_loop(0, P // LANES, _zero, 0)

    sel0 = jnp.zeros((LANES,), jnp.int32)
    sel1 = sel0 + 1
    sel2 = sel0 + 2
    sel3 = sel0 + 3

    def _row(r, _):
        b = base + r
        sbv = plsc.load_gather(a2s_v, [sel0 + r])
        y0 = plsc.load_gather(y_v, [r * L + iota])
        y1 = plsc.load_gather(y_v, [r * L + LANES + iota])
        idx_v[pl.ds(0, LANES)] = sbv * (P * 4 // 128) + (y0 >> 5)
        idx_v[pl.ds(LANES, LANES)] = sbv * (P * 4 // 128) + (y1 >> 5)
        pltpu.async_copy(params_hbm.at[idx_v], prow_v, gsem).wait()
        for h, yh in ((0, y0), (1, y1)):
            rows = iota + h * LANES
            off = (yh & 31) * 4
            c1 = plsc.load_gather(prow_v, [rows, off + 1])
            c2 = plsc.load_gather(prow_v, [rows, off + 2])
            c3 = plsc.load_gather(prow_v, [rows, off + 3])
            plsc.addupdate_scatter(bpre_v, [yh], c2 + c3)
            plsc.addupdate_scatter(badd_v, [yh], c1)
            plsc.addupdate_scatter(bdel_v, [yh], c3)
        cp0 = pltpu.async_copy(bpre_v, pre_hbm.at[b], osem)
        cp1 = pltpu.async_copy(badd_v, add_hbm.at[b], osem)
        cp2 = pltpu.async_copy(bdel_v, del_hbm.at[b], osem)
        cp0.wait()
        cp1.wait()
        cp2.wait()
        for yh in (y0, y1):
            plsc.store_scatter(bpre_v, [yh], zf)
            plsc.store_scatter(badd_v, [yh], zf)
            plsc.store_scatter(bdel_v, [yh], zf)
        return 0

    lax.fori_loop(0, ROWS, _row, 0)


@functools.partial(jax.jit, donate_argnums=())
def kernel(schema_params, y_indices, action_to_schema):
    params2d = schema_params.reshape(S * P * 4 // 128, 128)
    y_flat = y_indices.reshape(B * L)
    mesh = plsc.VectorSubcoreMesh(core_axis_name="c", subcore_axis_name="s")
    out = jax.ShapeDtypeStruct((B, P), jnp.float32)
    run = pl.kernel(
        _sc_body,
        out_type=[out, out, out],
        mesh=mesh,
        compiler_params=pltpu.CompilerParams(needs_layout_passes=False),
        scratch_types=[
            pltpu.VMEM((ROWS * L,), jnp.int32),
            pltpu.VMEM((ROWS,), jnp.int32),
            pltpu.VMEM((L,), jnp.int32),
            pltpu.VMEM((L, 128), jnp.float32),
            pltpu.VMEM((P,), jnp.float32),
            pltpu.VMEM((P,), jnp.float32),
            pltpu.VMEM((P,), jnp.float32),
            pltpu.SemaphoreType.DMA,
            pltpu.SemaphoreType.DMA,
        ],
    )
    pre, add, dele = run(params2d, y_flat, action_to_schema)
    return (pre, add, dele)


# speedup vs baseline: 3.4156x; 1.0846x over previous
import functools

import jax
import jax.numpy as jnp
from jax import lax
from jax.experimental import pallas as pl
from jax.experimental.pallas import tpu as pltpu
from jax.experimental.pallas import tpu_sc as plsc

B, P, S, L = 1024, 16384, 8, 32
NC, NS = 2, 16
NW = NC * NS
ROWS = B // NW
LANES = 16
GROWS = P * 4 // 128


def _sc_body(params_hbm, y_hbm, a2s_hbm, pre_hbm, add_hbm, del_hbm,
             y_v, a2s_v, idx0_v, idx1_v, prow0_v, prow1_v,
             bpre0, badd0, bdel0, bpre1, badd1, bdel1,
             gsem0, gsem1, osem0, osem1):
    wid = lax.axis_index("s") * NC + lax.axis_index("c")
    base = wid * ROWS

    pltpu.sync_copy(y_hbm.at[pl.ds(base * L, ROWS * L)], y_v)
    pltpu.sync_copy(a2s_hbm.at[pl.ds(base, ROWS)], a2s_v)

    zf = jnp.zeros((LANES,), jnp.float32)
    iota = lax.iota(jnp.int32, LANES)
    sel0 = jnp.zeros((LANES,), jnp.int32)

    bufs = ((bpre0, badd0, bdel0), (bpre1, badd1, bdel1))
    idxs = (idx0_v, idx1_v)
    prows = (prow0_v, prow1_v)
    gsems = (gsem0, gsem1)
    osems = (osem0, osem1)
    outs = (pre_hbm, add_hbm, del_hbm)

    def _zero(i, _):
        for j in range(4):
            off = (i * 4 + j) * LANES
            for bset in bufs:
                for bref in bset:
                    bref[pl.ds(off, LANES)] = zf
        return 0

    lax.fori_loop(0, P // LANES // 4, _zero, 0)

    def load_y(r):
        y0 = plsc.load_gather(y_v, [r * L + iota])
        y1 = plsc.load_gather(y_v, [r * L + LANES + iota])
        return y0, y1

    def _pair(g, _):
        for k in (0, 1):
            r = g * 2 + k
            b = base + r
            sbv = plsc.load_gather(a2s_v, [sel0 + r])
            y0, y1 = load_y(r)
            idxs[k][pl.ds(0, LANES)] = sbv * GROWS + (y0 >> 5)
            idxs[k][pl.ds(LANES, LANES)] = sbv * GROWS + (y1 >> 5)
            gcp = pltpu.async_copy(params_hbm.at[idxs[k]], prows[k], gsems[k])

            @pl.when(g > 0)
            def _():
                for bref in bufs[k]:
                    pltpu.make_async_copy(bref, pre_hbm.at[b - 2],
                                          osems[k]).wait()
                yp0, yp1 = load_y(r - 2)
                for bref in bufs[k]:
                    plsc.store_scatter(bref, [yp0], zf)
                    plsc.store_scatter(bref, [yp1], zf)

            gcp.wait()
            for h, yh in ((0, y0), (1, y1)):
                rows = iota + h * LANES
                off = (yh & 31) * 4
                c1 = plsc.load_gather(prows[k], [rows, off + 1])
                c2 = plsc.load_gather(prows[k], [rows, off + 2])
                c3 = plsc.load_gather(prows[k], [rows, off + 3])
                plsc.addupdate_scatter(bufs[k][0], [yh], c2 + c3)
                plsc.addupdate_scatter(bufs[k][1], [yh], c1)
                plsc.addupdate_scatter(bufs[k][2], [yh], c3)
            for bref, o in zip(bufs[k], outs):
                pltpu.async_copy(bref, o.at[b], osems[k])
        return 0

    lax.fori_loop(0, ROWS // 2, _pair, 0)

    for k in (0, 1):
        for bref, o in zip(bufs[k], outs):
            pltpu.make_async_copy(bref, o.at[base + ROWS - 2 + k],
                                  osems[k]).wait()


@functools.partial(jax.jit, donate_argnums=())
def kernel(schema_params, y_indices, action_to_schema):
    params2d = schema_params.reshape(GROWS * S, 128)
    y_flat = y_indices.reshape(B * L)
    mesh = plsc.VectorSubcoreMesh(core_axis_name="c", subcore_axis_name="s")
    out = jax.ShapeDtypeStruct((B, P), jnp.float32)
    run = pl.kernel(
        _sc_body,
        out_type=[out, out, out],
        mesh=mesh,
        compiler_params=pltpu.CompilerParams(needs_layout_passes=False),
        scratch_types=[
            pltpu.VMEM((ROWS * L,), jnp.int32),
            pltpu.VMEM((ROWS,), jnp.int32),
            pltpu.VMEM((L,), jnp.int32),
            pltpu.VMEM((L,), jnp.int32),
            pltpu.VMEM((L, 128), jnp.float32),
            pltpu.VMEM((L, 128), jnp.float32),
            pltpu.VMEM((P,), jnp.float32),
            pltpu.VMEM((P,), jnp.float32),
            pltpu.VMEM((P,), jnp.float32),
            pltpu.VMEM((P,), jnp.float32),
            pltpu.VMEM((P,), jnp.float32),
            pltpu.VMEM((P,), jnp.float32),
            pltpu.SemaphoreType.DMA,
            pltpu.SemaphoreType.DMA,
            pltpu.SemaphoreType.DMA,
            pltpu.SemaphoreType.DMA,
        ],
    )
    pre, add, dele = run(params2d, y_flat, action_to_schema)
    return (pre, add, dele)
